# Initial kernel scaffold; baseline (speedup 1.0000x reference)
import jax
import jax.numpy as jnp
from jax.experimental import pallas as pl


def kernel(x_cat, x_cont, median, factors):
    raise NotImplementedError("write your pallas kernel here")


# speedup vs baseline: 10.1714x; 10.1714x over previous
import functools

import jax
import jax.numpy as jnp
import numpy as np
from jax.experimental import pallas as pl

_CAT_DIMS = [100] * 26
_N_CAT = len(_CAT_DIMS)
_N_CONT = 13
_CAT_W = sum(_CAT_DIMS)
_OUT_W = _CAT_W + _N_CONT
_BR = 512


def _build_tables():
    feat = np.full((_OUT_W,), -1, dtype=np.int64)
    within = np.full((1, _OUT_W), -1.0, dtype=np.float32)
    off = 0
    for i, w in enumerate(_CAT_DIMS):
        feat[off:off + w] = i
        within[0, off:off + w] = np.arange(w, dtype=np.float32)
        off += w
    M = np.zeros((_N_CAT, _OUT_W), dtype=np.float32)
    cols = np.arange(_CAT_W)
    M[feat[:_CAT_W], cols] = 1.0
    P = np.zeros((_N_CONT, _OUT_W), dtype=np.float32)
    P[np.arange(_N_CONT), _CAT_W + np.arange(_N_CONT)] = 1.0
    return jnp.asarray(M), jnp.asarray(within), jnp.asarray(P)


def _body(x_cat_ref, x_cont_ref, med_ref, fac_ref, m_ref, within_ref, p_ref,
          out_ref):
    xc = x_cat_ref[...].astype(jnp.float32)
    g = jnp.dot(xc, m_ref[...], preferred_element_type=jnp.float32)
    oh = (g == within_ref[...]).astype(jnp.float32)

    xs = fac_ref[...] * (x_cont_ref[...] - med_ref[...])
    t = xs / jnp.sqrt(1.0 + (xs / 3.0) ** 2)
    out_ref[...] = oh + jnp.dot(t, p_ref[...], preferred_element_type=jnp.float32)


@jax.jit
def _run(x_cat, x_cont, median, factors, M, within, P):
    n = x_cat.shape[0]
    grid = (n // _BR,)
    return pl.pallas_call(
        _body,
        grid=grid,
        in_specs=[
            pl.BlockSpec((_BR, _N_CAT), lambda i: (i, 0)),
            pl.BlockSpec((_BR, _N_CONT), lambda i: (i, 0)),
            pl.BlockSpec((1, _N_CONT), lambda i: (0, 0)),
            pl.BlockSpec((1, _N_CONT), lambda i: (0, 0)),
            pl.BlockSpec((_N_CAT, _OUT_W), lambda i: (0, 0)),
            pl.BlockSpec((1, _OUT_W), lambda i: (0, 0)),
            pl.BlockSpec((_N_CONT, _OUT_W), lambda i: (0, 0)),
        ],
        out_specs=pl.BlockSpec((_BR, _OUT_W), lambda i: (i, 0)),
        out_shape=jax.ShapeDtypeStruct((n, _OUT_W), jnp.float32),
    )(x_cat, x_cont, median.reshape(1, -1), factors.reshape(1, -1), M, within, P)


def kernel(x_cat, x_cont, median, factors):
    M, within, P = _build_tables()
    return _run(x_cat.astype(jnp.int32), x_cont, median, factors, M, within, P)
